# baseline (speedup 1.0000x reference)
import jax
import jax.numpy as jnp
from jax.experimental import pallas as pl
from jax.experimental.pallas import tpu as pltpu


def _concat_body(idx_ref, w_ref, feat_ref, out_ref):
    out_ref[0, 0, :] = w_ref[0, 0]
    out_ref[0, 1:, :] = feat_ref[0]


def kernel(feature, index_value, embedding_offset, W):
    B, S, D = feature.shape
    idx = (index_value - embedding_offset).astype(jnp.int32)
    W3 = W.reshape(W.shape[0], 1, D)
    grid_spec = pltpu.PrefetchScalarGridSpec(
        num_scalar_prefetch=1,
        grid=(B,),
        in_specs=[
            pl.BlockSpec((1, 1, D), lambda b, idx_ref: (idx_ref[b], 0, 0)),
            pl.BlockSpec((1, S, D), lambda b, idx_ref: (b, 0, 0)),
        ],
        out_specs=pl.BlockSpec((1, S + 1, D), lambda b, idx_ref: (b, 0, 0)),
    )
    return pl.pallas_call(
        _concat_body,
        grid_spec=grid_spec,
        out_shape=jax.ShapeDtypeStruct((B, S + 1, D), feature.dtype),
    )(idx, W3, feature)


# speedup vs baseline: 1.0050x; 1.0050x over previous
import jax
import jax.numpy as jnp
from jax.experimental import pallas as pl
from jax.experimental.pallas import tpu as pltpu


def _concat_body(idx_ref, w_ref, feat_ref, out_ref):
    b = pl.program_id(0)
    out_ref[0, 0, :] = w_ref[idx_ref[b], :]
    out_ref[0, 1:, :] = feat_ref[0]


def kernel(feature, index_value, embedding_offset, W):
    B, S, D = feature.shape
    idx = (index_value - embedding_offset).astype(jnp.int32)
    V = W.shape[0]
    grid_spec = pltpu.PrefetchScalarGridSpec(
        num_scalar_prefetch=1,
        grid=(B,),
        in_specs=[
            pl.BlockSpec((V, D), lambda b, idx_ref: (0, 0)),
            pl.BlockSpec((1, S, D), lambda b, idx_ref: (b, 0, 0)),
        ],
        out_specs=pl.BlockSpec((1, S + 1, D), lambda b, idx_ref: (b, 0, 0)),
    )
    return pl.pallas_call(
        _concat_body,
        grid_spec=grid_spec,
        out_shape=jax.ShapeDtypeStruct((B, S + 1, D), feature.dtype),
    )(idx, W, feature)


# speedup vs baseline: 1.0098x; 1.0048x over previous
import jax
import jax.numpy as jnp
from jax.experimental import pallas as pl
from jax.experimental.pallas import tpu as pltpu


_BB = 2


def _concat_body(idx_ref, w_ref, feat_ref, out_ref):
    g = pl.program_id(0)
    for j in range(_BB):
        out_ref[j, 0, :] = w_ref[idx_ref[g * _BB + j], :]
        out_ref[j, 1:, :] = feat_ref[j]


def kernel(feature, index_value, embedding_offset, W):
    B, S, D = feature.shape
    idx = (index_value - embedding_offset).astype(jnp.int32)
    V = W.shape[0]
    grid_spec = pltpu.PrefetchScalarGridSpec(
        num_scalar_prefetch=1,
        grid=(B // _BB,),
        in_specs=[
            pl.BlockSpec((V, D), lambda b, idx_ref: (0, 0)),
            pl.BlockSpec((_BB, S, D), lambda b, idx_ref: (b, 0, 0)),
        ],
        out_specs=pl.BlockSpec((_BB, S + 1, D), lambda b, idx_ref: (b, 0, 0)),
    )
    return pl.pallas_call(
        _concat_body,
        grid_spec=grid_spec,
        out_shape=jax.ShapeDtypeStruct((B, S + 1, D), feature.dtype),
    )(idx, W, feature)
